# baseline (speedup 1.0000x reference)
import functools

import jax
import jax.numpy as jnp
from jax import lax
from jax.experimental import pallas as pl
from jax.experimental.pallas import tpu as pltpu
from jax.experimental.pallas import tpu_sc as plsc

USERS = 10000
ITEMS = 40000
N = USERS + ITEMS
E = 800000
D = 64
B = 1024
N_LAYERS = 3

NC = 2
NS = 16
HALF = N // NC
ZCH = 80
NZCH = (HALF + ZCH - 1) // ZCH
ACC_ROWS = NZCH * ZCH
DUMMY = HALF

SUB = 400
NCHUNK = E // SUB // NS
EROW = 3 * SUB
DR = 200
NDR = HALF // DR


def _propagate_body(edata_hbm, emb_hbm, out_hbm,
                    acc, ebuf, dloc, gbuf, isem, gsem, ssem):
    c = lax.axis_index("c")
    s = lax.axis_index("s")
    base = c * HALF

    def zfill(r, carry):
        for cc in range(2):
            gbuf[0, r, pl.ds(cc * 32, 32)] = jnp.zeros((32,), jnp.bfloat16)
        return carry
    lax.fori_loop(0, ZCH, zfill, 0)

    def zcopy(j, carry):
        idx = s + j * NS

        @pl.when(idx < NZCH)
        def _():
            pltpu.sync_copy(gbuf.at[0, pl.ds(0, ZCH)],
                            acc.at[pl.ds(idx * ZCH, ZCH)])
        return carry
    lax.fori_loop(0, (NZCH + NS - 1) // NS, zcopy, 0)
    plsc.subcore_barrier()

    def _idx_copy(i, b):
        eoff = (s * NCHUNK + i) * EROW
        return (edata_hbm.at[pl.ds(eoff, EROW)], ebuf.at[b])

    sr0, dr0 = _idx_copy(0, 0)
    pltpu.async_copy(sr0, dr0, isem)

    def _gather_pair(b):
        return (emb_hbm.at[ebuf.at[b, pl.ds(0, SUB)]], gbuf.at[b])

    def _scale_and_scatter(nb):
        def scale(g, carry):
            wv16 = plsc.bitcast(ebuf[nb, pl.ds(2 * SUB + g * 16, 16)],
                                jnp.float32)
            for lane in range(16):
                wf = jnp.full((16,), wv16[lane], jnp.float32)
                wsp = plsc.pack(wf, wf, format=plsc.PackFormat.INTERLEAVED)
                jj = g * 16 + lane
                for cc in range(2):
                    gbuf[nb, jj, pl.ds(cc * 32, 32)] = (
                        gbuf[nb, jj, pl.ds(cc * 32, 32)] * wsp)
            return carry
        lax.fori_loop(0, SUB // 16, scale, 0)
        pltpu.async_copy(gbuf.at[nb], acc.at[dloc.at[nb]], ssem.at[nb],
                         add=True)

    def chunk(i, carry):
        b = lax.rem(i, 2)
        nb = 1 - b

        @pl.when(i < NCHUNK)
        def _():
            sr, dr = _idx_copy(i, b)
            pltpu.make_async_copy(sr, dr, isem).wait()

            @pl.when(i + 1 < NCHUNK)
            def _():
                sr2, dr2 = _idx_copy(i + 1, nb)
                pltpu.async_copy(sr2, dr2, isem)

            @pl.when(i >= 2)
            def _():
                pltpu.make_async_copy(gbuf.at[b], acc.at[dloc.at[b]],
                                      ssem.at[b]).wait()

            gsr, gdr = _gather_pair(b)
            pltpu.async_copy(gsr, gdr, gsem.at[b])

            for t in range(SUB // 16):
                d = ebuf[b, pl.ds(SUB + t * 16, 16)]
                loc = d - base
                ok = (loc >= 0) & (loc < HALF)
                dloc[b, pl.ds(t * 16, 16)] = jnp.where(ok, loc, DUMMY)

        @pl.when(i >= 1)
        def _():
            gsr, gdr = _gather_pair(nb)
            pltpu.make_async_copy(gsr, gdr, gsem.at[nb]).wait()
            _scale_and_scatter(nb)
        return carry
    lax.fori_loop(0, NCHUNK + 1, chunk, 0)
    for b in range(2):
        pltpu.make_async_copy(gbuf.at[b], acc.at[dloc.at[b]],
                              ssem.at[b]).wait()
    plsc.subcore_barrier()

    def drain(j, carry):
        idx = s + j * NS

        @pl.when(idx < NDR)
        def _():
            pltpu.sync_copy(acc.at[pl.ds(idx * DR, DR)],
                            out_hbm.at[pl.ds(base + idx * DR, DR)])
        return carry
    lax.fori_loop(0, (NDR + NS - 1) // NS, drain, 0)


def _user_gather_body(e1, e2, e3, uid_hbm, out_hbm, idx, r1, r2, r3, sem):
    c = lax.axis_index("c")
    s = lax.axis_index("s")
    w = s * NC + c
    rows = B // (NC * NS)
    base = w * rows
    pltpu.sync_copy(uid_hbm.at[pl.ds(base, rows)], idx)
    d1 = pltpu.async_copy(e1.at[idx], r1, sem)
    d2 = pltpu.async_copy(e2.at[idx], r2, sem)
    d3 = pltpu.async_copy(e3.at[idx], r3, sem)
    d1.wait(); d2.wait(); d3.wait()
    for r in range(rows):
        for cc in range(2):
            sl = pl.ds(cc * 32, 32)
            r1[r, sl] = r1[r, sl] + r2[r, sl] + r3[r, sl]
    pltpu.sync_copy(r1, out_hbm.at[pl.ds(base, rows)])


def _mm_body(u_ref, i1_ref, i2_ref, i3_ref, o_ref):
    its = (i1_ref[...].astype(jnp.float32) + i2_ref[...].astype(jnp.float32)
           + i3_ref[...].astype(jnp.float32)).astype(jnp.bfloat16)
    logits = lax.dot_general(u_ref[...], its, (((1,), (1,)), ((), ())),
                             preferred_element_type=jnp.float32)
    x = logits * (1.0 / 9.0)
    o_ref[...] = 1.0 / (1.0 + jnp.exp(-x))


_MESH = plsc.VectorSubcoreMesh(core_axis_name="c", subcore_axis_name="s")

_SC_PARAMS = pltpu.CompilerParams(use_tc_tiling_on_sc=False,
                                  needs_layout_passes=False)

_propagate = pl.kernel(
    _propagate_body,
    out_type=jax.ShapeDtypeStruct((N, D), jnp.bfloat16),
    mesh=_MESH,
    compiler_params=_SC_PARAMS,
    scratch_types=[
        pltpu.VMEM_SHARED((ACC_ROWS, D), jnp.bfloat16),
        pltpu.VMEM((2, EROW), jnp.int32),
        pltpu.VMEM((2, SUB), jnp.int32),
        pltpu.VMEM((2, SUB, D), jnp.bfloat16),
        pltpu.SemaphoreType.DMA,
        pltpu.SemaphoreType.DMA((2,)),
        pltpu.SemaphoreType.DMA((2,)),
    ],
)

_user_gather = pl.kernel(
    _user_gather_body,
    out_type=jax.ShapeDtypeStruct((B, D), jnp.bfloat16),
    mesh=_MESH,
    compiler_params=_SC_PARAMS,
    scratch_types=[
        pltpu.VMEM((B // (NC * NS),), jnp.int32),
        pltpu.VMEM((B // (NC * NS), D), jnp.bfloat16),
        pltpu.VMEM((B // (NC * NS), D), jnp.bfloat16),
        pltpu.VMEM((B // (NC * NS), D), jnp.bfloat16),
        pltpu.SemaphoreType.DMA,
    ],
)

_IB = 1024

_matmul = pl.pallas_call(
    _mm_body,
    grid=(pl.cdiv(ITEMS, _IB),),
    in_specs=[
        pl.BlockSpec((B, D), lambda i: (0, 0)),
        pl.BlockSpec((_IB, D), lambda i: (i, 0)),
        pl.BlockSpec((_IB, D), lambda i: (i, 0)),
        pl.BlockSpec((_IB, D), lambda i: (i, 0)),
    ],
    out_specs=pl.BlockSpec((B, _IB), lambda i: (0, i)),
    out_shape=jax.ShapeDtypeStruct((B, ITEMS), jnp.float32),
)


def kernel(user_ids, edge_index, edge_weight, user_table, item_table):
    nch = E // SUB
    edata = jnp.concatenate([
        edge_index[0].reshape(nch, 1, SUB),
        edge_index[1].reshape(nch, 1, SUB),
        lax.bitcast_convert_type(edge_weight, jnp.int32).reshape(nch, 1, SUB),
    ], axis=1).reshape(-1)
    emb = jnp.concatenate([user_table, item_table],
                          axis=0).astype(jnp.bfloat16)

    e1 = _propagate(edata, emb)
    e2 = _propagate(edata, e1)
    e3 = _propagate(edata, e2)

    u_sum = _user_gather(e1, e2, e3, user_ids)
    return _matmul(u_sum, e1[USERS:], e2[USERS:], e3[USERS:])
